# baseline (speedup 1.0000x reference)
import functools

import jax
import jax.numpy as jnp
from jax import lax
from jax.experimental import pallas as pl
from jax.experimental.pallas import tpu as pltpu
from jax.experimental.pallas import tpu_sc as plsc

N = 100000
D = 128
ROWS_PER_DEG = 16000
NUM_WORKERS = 32
IW = 128
_PARAMS = {1: (128, 4), 2: (64, 8), 3: (32, 16), 4: (32, 16),
           5: (16, 32), 6: (16, 32)}
_SECBASE = {1: 0, 2: 512, 3: 1536, 4: 3072, 5: 5120, 6: 7680}
_IDXWORDS = 10752
_DEGOFF = {1: 0, 2: 16000, 3: 48000, 4: 96000, 5: 160000, 6: 240000}


def _reduce(d, R, g, ob):
    def row_body(r, carry):
        base = r * d
        for cc in range(D // 16):
            sl = pl.ds(cc * 16, 16)
            v = g[base, sl]
            for j in range(1, d):
                v = v + g[base + j, sl]
            ob[r, sl] = v
        return carry
    lax.fori_loop(0, R, row_body, 0, unroll=2 if d < 4 else 1)


def _sc_gather_sum(nf, adj_flat):
    mesh = plsc.VectorSubcoreMesh(core_axis_name="c", subcore_axis_name="s")

    @functools.partial(
        pl.kernel,
        out_type=jax.ShapeDtypeStruct((6 * ROWS_PER_DEG, D), jnp.float32),
        mesh=mesh,
        scratch_types=[
            pltpu.VMEM((_IDXWORDS,), jnp.int32),
            pltpu.VMEM((IW, D), jnp.float32),
            pltpu.VMEM((IW, D), jnp.float32),
            pltpu.VMEM((IW, D), jnp.float32),
            pltpu.VMEM((IW, D), jnp.float32),
            pltpu.VMEM((IW, D), jnp.float32),
            pltpu.VMEM((IW, D), jnp.float32),
            pltpu.SemaphoreType.DMA,
            pltpu.SemaphoreType.DMA,
            pltpu.SemaphoreType.DMA,
            pltpu.SemaphoreType.DMA,
            pltpu.SemaphoreType.DMA,
            pltpu.SemaphoreType.DMA,
            pltpu.SemaphoreType.DMA,
        ],
    )
    def k(nf_hbm, adj_hbm, out_hbm, idx_v, g0, g1, g2, g3, o0, o1,
          sem_idx, gs0, gs1, gs2, gs3, os0, os1):
        cid = lax.axis_index("c")
        sid = lax.axis_index("s")
        wid = sid * 2 + cid
        gbufs, gsems = (g0, g1, g2, g3), (gs0, gs1, gs2, gs3)
        obufs, osems = (o0, o1), (os0, os1)

        w0s = {}
        slab_cps = []
        for d in range(1, 7):
            R, S = _PARAMS[d]
            w0s[d] = jnp.minimum(wid * S * R, ROWS_PER_DEG - S * R)
            n = S * R * d
            slab_cps.append(pltpu.async_copy(
                adj_hbm.at[pl.ds(_DEGOFF[d] + w0s[d] * d, n)],
                idx_v.at[pl.ds(_SECBASE[d], n)], sem_idx))
        for cp in slab_cps:
            cp.wait()

        def base_row(d, s):
            R, S = _PARAMS[d]
            return jnp.minimum((wid * S + s) * R, ROWS_PER_DEG - R)

        def idx_slice(d, s):
            R, S = _PARAMS[d]
            off = _SECBASE[d] + (base_row(d, s) - w0s[d]) * d
            return idx_v.at[pl.ds(off, R * d)]

        def store_row0(d, s):
            return (d - 1) * ROWS_PER_DEG + base_row(d, s)

        R1, S1 = _PARAMS[1]
        cps = [pltpu.async_copy(nf_hbm.at[idx_slice(1, s)], gbufs[s],
                                gsems[s])
               for s in range(S1)]
        sts = []
        for s in range(S1):
            cps[s].wait()
            sts.append(pltpu.async_copy(
                gbufs[s], out_hbm.at[pl.ds(store_row0(1, s), R1)], gsems[s]))
        for s in range(S1):
            sts[s].wait()

        for d in range(2, 7):
            R, S = _PARAMS[d]
            L = R * d
            NB = 4 if d < 5 else 2

            def fire_gather(s, b, d=d, L=L):
                return pltpu.async_copy(
                    nf_hbm.at[idx_slice(d, s)],
                    gbufs[b].at[pl.ds(0, L)], gsems[b])

            def task(s, b, first, last, d=d, R=R, L=L):
                pltpu.make_async_copy(
                    nf_hbm.at[pl.ds(0, L)], gbufs[b].at[pl.ds(0, L)],
                    gsems[b]).wait()
                if not first:
                    pltpu.make_async_copy(
                        obufs[b % 2].at[pl.ds(0, R)],
                        nf_hbm.at[pl.ds(0, R)], osems[b % 2]).wait()
                _reduce(d, R, gbufs[b], obufs[b % 2])
                pltpu.async_copy(
                    obufs[b % 2].at[pl.ds(0, R)],
                    out_hbm.at[pl.ds(store_row0(d, s), R)], osems[b % 2])
                if not last:
                    fire_gather(s + NB, b)

            for b in range(NB):
                fire_gather(b, b)
            for b in range(NB):
                task(b, b, first=(b < 2), last=False)

            def outer(o, carry, task=task, NB=NB):
                s0 = NB * o
                for b in range(NB):
                    task(s0 + b, b, first=False, last=False)
                return carry
            lax.fori_loop(1, S // NB - 1, outer, 0)

            for b in range(NB):
                task(S - NB + b, b, first=False, last=True)
            for b in range(2):
                pltpu.make_async_copy(
                    obufs[b].at[pl.ds(0, R)],
                    nf_hbm.at[pl.ds(0, R)], osems[b]).wait()

    return k(nf, adj_flat)


BS = 4000


def _tc_self(nf, W, b):
    nblocks = N // BS

    def ws_idx(g):
        return (jnp.where(g == 0, 0, 2 * ((g + 3) // 4)), 0, 0)

    def body(x_ref, ws_ref, bs_ref, o_ref):
        o_ref[...] = jnp.dot(
            x_ref[...], ws_ref[0],
            preferred_element_type=jnp.float32) + bs_ref[0, 0]

    br = b.reshape(b.shape[0], 1, D)
    return pl.pallas_call(
        body,
        grid=(nblocks,),
        in_specs=[
            pl.BlockSpec((BS, D), lambda g: (g, 0)),
            pl.BlockSpec((1, D, D), ws_idx),
            pl.BlockSpec((1, 1, D), ws_idx),
        ],
        out_specs=pl.BlockSpec((BS, D), lambda g: (g, 0)),
        out_shape=jax.ShapeDtypeStruct((N, D), jnp.float32),
    )(nf, W, br)


def _tc_add_neigh(out_self, nsum, W, b):
    nblocks = 6 * ROWS_PER_DEG // BS

    def wn_idx(g):
        return (2 * ((g + 4) // 4) - 1, 0, 0)

    def body(prev_ref, ns_ref, wn_ref, bn_ref, o_ref):
        o_ref[...] = prev_ref[...] + jnp.dot(
            ns_ref[...], wn_ref[0],
            preferred_element_type=jnp.float32) + bn_ref[0, 0]

    br = b.reshape(b.shape[0], 1, D)
    return pl.pallas_call(
        body,
        grid=(nblocks,),
        in_specs=[
            pl.BlockSpec((BS, D), lambda g: (g + 1, 0)),
            pl.BlockSpec((BS, D), lambda g: (g, 0)),
            pl.BlockSpec((1, D, D), wn_idx),
            pl.BlockSpec((1, 1, D), wn_idx),
        ],
        out_specs=pl.BlockSpec((BS, D), lambda g: (g + 1, 0)),
        out_shape=jax.ShapeDtypeStruct((N, D), jnp.float32),
        input_output_aliases={0: 0},
    )(out_self, nsum, W, br)


def kernel(node_features, deg_slice, deg_adj_1, deg_adj_2, deg_adj_3,
           deg_adj_4, deg_adj_5, deg_adj_6, W, b):
    adjs = (deg_adj_1, deg_adj_2, deg_adj_3, deg_adj_4, deg_adj_5, deg_adj_6)
    flats = [a if a.dtype == jnp.int32 else a.astype(jnp.int32) for a in adjs]
    adj_flat = jnp.concatenate([f.reshape(-1) for f in flats])
    out_self = _tc_self(node_features, W, b)
    nsum = _sc_gather_sum(node_features, adj_flat)
    return _tc_add_neigh(out_self, nsum, W, b)


# speedup vs baseline: 1.0555x; 1.0555x over previous
import functools

import jax
import jax.numpy as jnp
from jax import lax
from jax.experimental import pallas as pl
from jax.experimental.pallas import tpu as pltpu
from jax.experimental.pallas import tpu_sc as plsc

N = 100000
D = 128
ROWS_PER_DEG = 16000
NUM_WORKERS = 32
IW = 128
BS = 4000
_PARAMS = {1: (128, 4), 2: (64, 8), 3: (40, 14), 4: (32, 16),
           5: (24, 22), 6: (16, 32)}
_PART_A = (1, 2, 3)
_PART_B = (4, 5, 6)


def _reduce(d, R, g, ob):
    def row_body(r, carry):
        base = r * d
        for cc in range(D // 16):
            sl = pl.ds(cc * 16, 16)
            v = g[base, sl]
            for j in range(1, d):
                v = v + g[base + j, sl]
            ob[r, sl] = v
        return carry
    lax.fori_loop(0, R, row_body, 0, unroll=2 if d < 4 else 1)


def _sc_gather_sum(nf, adj_flat, degs):
    mesh = plsc.VectorSubcoreMesh(core_axis_name="c", subcore_axis_name="s")

    degoff, secbase = {}, {}
    ao = so = 0
    for d in degs:
        R, S = _PARAMS[d]
        degoff[d], secbase[d] = ao, so
        ao += ROWS_PER_DEG * d
        so += S * R * d
    idxwords = so

    @functools.partial(
        pl.kernel,
        out_type=jax.ShapeDtypeStruct((len(degs) * ROWS_PER_DEG, D),
                                      jnp.float32),
        mesh=mesh,
        scratch_types=[
            pltpu.VMEM((idxwords,), jnp.int32),
            pltpu.VMEM((IW, D), jnp.float32),
            pltpu.VMEM((IW, D), jnp.float32),
            pltpu.VMEM((IW, D), jnp.float32),
            pltpu.VMEM((IW, D), jnp.float32),
            pltpu.SemaphoreType.DMA,
            pltpu.SemaphoreType.DMA,
            pltpu.SemaphoreType.DMA,
            pltpu.SemaphoreType.DMA,
            pltpu.SemaphoreType.DMA,
        ],
    )
    def k(nf_hbm, adj_hbm, out_hbm, idx_v, g0, g1, o0, o1,
          sem_idx, gs0, gs1, os0, os1):
        cid = lax.axis_index("c")
        sid = lax.axis_index("s")
        wid = sid * 2 + cid
        gbufs, gsems = (g0, g1), (gs0, gs1)
        obufs, osems = (o0, o1), (os0, os1)

        w0s = {}
        slab_cps = []
        for d in degs:
            R, S = _PARAMS[d]
            w0s[d] = jnp.minimum(wid * S * R, ROWS_PER_DEG - S * R)
            n = S * R * d
            slab_cps.append(pltpu.async_copy(
                adj_hbm.at[pl.ds(degoff[d] + w0s[d] * d, n)],
                idx_v.at[pl.ds(secbase[d], n)], sem_idx))
        for cp in slab_cps:
            cp.wait()

        def base_row(d, s):
            R, S = _PARAMS[d]
            return jnp.minimum((wid * S + s) * R, ROWS_PER_DEG - R)

        def idx_slice(d, s):
            R, S = _PARAMS[d]
            off = secbase[d] + (base_row(d, s) - w0s[d]) * d
            return idx_v.at[pl.ds(off, R * d)]

        def store_row0(d, s):
            return degs.index(d) * ROWS_PER_DEG + base_row(d, s)

        for d in degs:
            R, S = _PARAMS[d]
            L = R * d

            if d == 1:
                bufs4 = (g0, g1, o0, o1)
                sems4 = (gs0, gs1, os0, os1)
                cps = [pltpu.async_copy(nf_hbm.at[idx_slice(1, s)],
                                        bufs4[s], sems4[s])
                       for s in range(S)]
                sts = []
                for s in range(S):
                    cps[s].wait()
                    sts.append(pltpu.async_copy(
                        bufs4[s], out_hbm.at[pl.ds(store_row0(1, s), R)],
                        sems4[s]))
                for s in range(S):
                    sts[s].wait()
                continue

            def fire_gather(s, b, d=d, L=L):
                return pltpu.async_copy(
                    nf_hbm.at[idx_slice(d, s)],
                    gbufs[b].at[pl.ds(0, L)], gsems[b])

            def task(s, b, first, last, d=d, R=R, L=L):
                pltpu.make_async_copy(
                    nf_hbm.at[pl.ds(0, L)], gbufs[b].at[pl.ds(0, L)],
                    gsems[b]).wait()
                if not first:
                    pltpu.make_async_copy(
                        obufs[b].at[pl.ds(0, R)],
                        nf_hbm.at[pl.ds(0, R)], osems[b]).wait()
                _reduce(d, R, gbufs[b], obufs[b])
                pltpu.async_copy(
                    obufs[b].at[pl.ds(0, R)],
                    out_hbm.at[pl.ds(store_row0(d, s), R)], osems[b])
                if not last:
                    fire_gather(s + 2, b)

            fire_gather(0, 0)
            fire_gather(1, 1)
            task(0, 0, first=True, last=False)
            task(1, 1, first=True, last=False)

            def outer(o, carry, task=task):
                s0 = 2 * o
                task(s0, 0, first=False, last=False)
                task(s0 + 1, 1, first=False, last=False)
                return carry
            lax.fori_loop(1, S // 2 - 1, outer, 0)

            task(S - 2, 0, first=False, last=True)
            task(S - 1, 1, first=False, last=True)
            for b in range(2):
                pltpu.make_async_copy(
                    obufs[b].at[pl.ds(0, R)],
                    nf_hbm.at[pl.ds(0, R)], osems[b]).wait()

    return k(nf, adj_flat)


def _tc_self(nf, W, b):
    nblocks = N // BS

    def ws_idx(g):
        return (jnp.where(g == 0, 0, 2 * ((g + 3) // 4)), 0, 0)

    def body(x_ref, ws_ref, bs_ref, o_ref):
        o_ref[...] = jnp.dot(
            x_ref[...], ws_ref[0],
            preferred_element_type=jnp.float32) + bs_ref[0, 0]

    br = b.reshape(b.shape[0], 1, D)
    return pl.pallas_call(
        body,
        grid=(nblocks,),
        in_specs=[
            pl.BlockSpec((BS, D), lambda g: (g, 0)),
            pl.BlockSpec((1, D, D), ws_idx),
            pl.BlockSpec((1, 1, D), ws_idx),
        ],
        out_specs=pl.BlockSpec((BS, D), lambda g: (g, 0)),
        out_shape=jax.ShapeDtypeStruct((N, D), jnp.float32),
    )(nf, W, br)


def _tc_add_neigh(out_self, nsum_a, nsum_b, W, b):
    nblocks = 6 * ROWS_PER_DEG // BS
    half = len(_PART_A) * ROWS_PER_DEG // BS

    def wn_idx(g):
        return (2 * ((g + 4) // 4) - 1, 0, 0)

    def body(prev_ref, nsa_ref, nsb_ref, wn_ref, bn_ref, o_ref):
        g = pl.program_id(0)

        @pl.when(g < half)
        def _():
            o_ref[...] = prev_ref[...] + jnp.dot(
                nsa_ref[...], wn_ref[0],
                preferred_element_type=jnp.float32) + bn_ref[0, 0]

        @pl.when(g >= half)
        def _():
            o_ref[...] = prev_ref[...] + jnp.dot(
                nsb_ref[...], wn_ref[0],
                preferred_element_type=jnp.float32) + bn_ref[0, 0]

    br = b.reshape(b.shape[0], 1, D)
    return pl.pallas_call(
        body,
        grid=(nblocks,),
        in_specs=[
            pl.BlockSpec((BS, D), lambda g: (g + 1, 0)),
            pl.BlockSpec((BS, D), lambda g: (jnp.minimum(g, half - 1), 0)),
            pl.BlockSpec((BS, D), lambda g: (jnp.maximum(g - half, 0), 0)),
            pl.BlockSpec((1, D, D), wn_idx),
            pl.BlockSpec((1, 1, D), wn_idx),
        ],
        out_specs=pl.BlockSpec((BS, D), lambda g: (g + 1, 0)),
        out_shape=jax.ShapeDtypeStruct((N, D), jnp.float32),
        input_output_aliases={0: 0},
    )(out_self, nsum_a, nsum_b, W, br)


def kernel(node_features, deg_slice, deg_adj_1, deg_adj_2, deg_adj_3,
           deg_adj_4, deg_adj_5, deg_adj_6, W, b):
    adjs = {1: deg_adj_1, 2: deg_adj_2, 3: deg_adj_3,
            4: deg_adj_4, 5: deg_adj_5, 6: deg_adj_6}

    def flat(degs):
        parts = [adjs[d] if adjs[d].dtype == jnp.int32
                 else adjs[d].astype(jnp.int32) for d in degs]
        return jnp.concatenate([p.reshape(-1) for p in parts])

    out_self = _tc_self(node_features, W, b)
    nsum_a = _sc_gather_sum(node_features, flat(_PART_A), _PART_A)
    nsum_b = _sc_gather_sum(node_features, flat(_PART_B), _PART_B)
    return _tc_add_neigh(out_self, nsum_a, nsum_b, W, b)


# speedup vs baseline: 1.0978x; 1.0401x over previous
import functools

import jax
import jax.numpy as jnp
from jax import lax
from jax.experimental import pallas as pl
from jax.experimental.pallas import tpu as pltpu
from jax.experimental.pallas import tpu_sc as plsc

N = 100000
D = 128
ROWS_PER_DEG = 16000
NUM_WORKERS = 32
IW = 128
BS = 4000
_PARAMS = {1: (128, 4), 2: (64, 8), 3: (40, 14), 4: (32, 16),
           5: (24, 22), 6: (16, 32)}
_PART_A = (1, 2, 3)
_PART_B = (4, 5, 6)


def _reduce(d, R, g, ob):
    def row_body(r, carry):
        base = r * d
        for cc in range(D // 16):
            sl = pl.ds(cc * 16, 16)
            v = g[base, sl]
            for j in range(1, d):
                v = v + g[base + j, sl]
            ob[r, sl] = v
        return carry
    lax.fori_loop(0, R, row_body, 0, unroll=2 if d < 4 else 1)


def _sc_gather_sum(nf, adj_flat, degs):
    mesh = plsc.VectorSubcoreMesh(core_axis_name="c", subcore_axis_name="s")

    degoff, secbase = {}, {}
    ao = so = 0
    for d in degs:
        R, S = _PARAMS[d]
        degoff[d], secbase[d] = ao, so
        ao += ROWS_PER_DEG * d
        so += S * R * d
    idxwords = so

    @functools.partial(
        pl.kernel,
        out_type=jax.ShapeDtypeStruct((len(degs) * ROWS_PER_DEG, D),
                                      jnp.float32),
        mesh=mesh,
        scratch_types=[
            pltpu.VMEM((idxwords,), jnp.int32),
            pltpu.VMEM((IW, D), jnp.float32),
            pltpu.VMEM((IW, D), jnp.float32),
            pltpu.VMEM((IW, D), jnp.float32),
            pltpu.VMEM((IW, D), jnp.float32),
            pltpu.SemaphoreType.DMA,
            pltpu.SemaphoreType.DMA,
            pltpu.SemaphoreType.DMA,
            pltpu.SemaphoreType.DMA,
            pltpu.SemaphoreType.DMA,
        ],
    )
    def k(nf_hbm, adj_hbm, out_hbm, idx_v, g0, g1, o0, o1,
          sem_idx, gs0, gs1, os0, os1):
        cid = lax.axis_index("c")
        sid = lax.axis_index("s")
        wid = sid * 2 + cid
        gbufs, gsems = (g0, g1), (gs0, gs1)
        obufs, osems = (o0, o1), (os0, os1)

        w0s = {}
        slab_cps = []
        for d in degs:
            R, S = _PARAMS[d]
            w0s[d] = jnp.minimum(wid * S * R, ROWS_PER_DEG - S * R)
            n = S * R * d
            slab_cps.append(pltpu.async_copy(
                adj_hbm.at[pl.ds(degoff[d] + w0s[d] * d, n)],
                idx_v.at[pl.ds(secbase[d], n)], sem_idx))
        for cp in slab_cps:
            cp.wait()

        def base_row(d, s):
            R, S = _PARAMS[d]
            return jnp.minimum((wid * S + s) * R, ROWS_PER_DEG - R)

        def idx_slice(d, s):
            R, S = _PARAMS[d]
            off = secbase[d] + (base_row(d, s) - w0s[d]) * d
            return idx_v.at[pl.ds(off, R * d)]

        def store_row0(d, s):
            return degs.index(d) * ROWS_PER_DEG + base_row(d, s)

        for d in degs:
            R, S = _PARAMS[d]
            L = R * d

            if d == 1:
                bufs4 = (g0, g1, o0, o1)
                sems4 = (gs0, gs1, os0, os1)
                cps = [pltpu.async_copy(nf_hbm.at[idx_slice(1, s)],
                                        bufs4[s], sems4[s])
                       for s in range(S)]
                sts = []
                for s in range(S):
                    cps[s].wait()
                    sts.append(pltpu.async_copy(
                        bufs4[s], out_hbm.at[pl.ds(store_row0(1, s), R)],
                        sems4[s]))
                for s in range(S):
                    sts[s].wait()
                continue

            def fire_gather(s, b, d=d, L=L):
                return pltpu.async_copy(
                    nf_hbm.at[idx_slice(d, s)],
                    gbufs[b].at[pl.ds(0, L)], gsems[b])

            def task(s, b, first, last, d=d, R=R, L=L):
                pltpu.make_async_copy(
                    nf_hbm.at[pl.ds(0, L)], gbufs[b].at[pl.ds(0, L)],
                    gsems[b]).wait()
                if not first:
                    pltpu.make_async_copy(
                        obufs[b].at[pl.ds(0, R)],
                        nf_hbm.at[pl.ds(0, R)], osems[b]).wait()
                _reduce(d, R, gbufs[b], obufs[b])
                pltpu.async_copy(
                    obufs[b].at[pl.ds(0, R)],
                    out_hbm.at[pl.ds(store_row0(d, s), R)], osems[b])
                if not last:
                    fire_gather(s + 2, b)

            fire_gather(0, 0)
            fire_gather(1, 1)
            task(0, 0, first=True, last=False)
            task(1, 1, first=True, last=False)

            def outer(o, carry, task=task):
                s0 = 2 * o
                task(s0, 0, first=False, last=False)
                task(s0 + 1, 1, first=False, last=False)
                return carry
            lax.fori_loop(1, S // 2 - 1, outer, 0)

            task(S - 2, 0, first=False, last=True)
            task(S - 1, 1, first=False, last=True)
            for b in range(2):
                pltpu.make_async_copy(
                    obufs[b].at[pl.ds(0, R)],
                    nf_hbm.at[pl.ds(0, R)], osems[b]).wait()

    return k(nf, adj_flat)


def _tc_self(nf, W, b):
    nblocks = N // BS

    def ws_idx(g):
        return (jnp.where(g == 0, 0, 2 * ((g + 3) // 4)), 0, 0)

    def body(x_ref, ws_ref, bs_ref, o_ref):
        o_ref[...] = jnp.dot(
            x_ref[...], ws_ref[0],
            preferred_element_type=jnp.float32) + bs_ref[0, 0]

    br = b.reshape(b.shape[0], 1, D)
    return pl.pallas_call(
        body,
        grid=(nblocks,),
        in_specs=[
            pl.BlockSpec((BS, D), lambda g: (g, 0)),
            pl.BlockSpec((1, D, D), ws_idx),
            pl.BlockSpec((1, 1, D), ws_idx),
        ],
        out_specs=pl.BlockSpec((BS, D), lambda g: (g, 0)),
        out_shape=jax.ShapeDtypeStruct((N, D), jnp.float32),
    )(nf, W, br)


def _tc_add_neigh(prev, nsum, W, b, blk0):
    nblocks = nsum.shape[0] // BS

    def wn_idx(g):
        return (2 * ((g + blk0 + 3) // 4) - 1, 0, 0)

    def body(prev_ref, ns_ref, wn_ref, bn_ref, o_ref):
        o_ref[...] = prev_ref[...] + jnp.dot(
            ns_ref[...], wn_ref[0],
            preferred_element_type=jnp.float32) + bn_ref[0, 0]

    br = b.reshape(b.shape[0], 1, D)
    return pl.pallas_call(
        body,
        grid=(nblocks,),
        in_specs=[
            pl.BlockSpec((BS, D), lambda g: (g + blk0, 0)),
            pl.BlockSpec((BS, D), lambda g: (g, 0)),
            pl.BlockSpec((1, D, D), wn_idx),
            pl.BlockSpec((1, 1, D), wn_idx),
        ],
        out_specs=pl.BlockSpec((BS, D), lambda g: (g + blk0, 0)),
        out_shape=jax.ShapeDtypeStruct((N, D), jnp.float32),
        input_output_aliases={0: 0},
    )(prev, nsum, W, br)


def kernel(node_features, deg_slice, deg_adj_1, deg_adj_2, deg_adj_3,
           deg_adj_4, deg_adj_5, deg_adj_6, W, b):
    adjs = {1: deg_adj_1, 2: deg_adj_2, 3: deg_adj_3,
            4: deg_adj_4, 5: deg_adj_5, 6: deg_adj_6}

    def flat(degs):
        parts = [adjs[d] if adjs[d].dtype == jnp.int32
                 else adjs[d].astype(jnp.int32) for d in degs]
        return jnp.concatenate([p.reshape(-1) for p in parts])

    out_self = _tc_self(node_features, W, b)
    nsum_a = _sc_gather_sum(node_features, flat(_PART_A), _PART_A)
    nsum_b = _sc_gather_sum(node_features, flat(_PART_B), _PART_B)
    out = _tc_add_neigh(out_self, nsum_b, W, b, blk0=13)
    return _tc_add_neigh(out, nsum_a, W, b, blk0=1)


# speedup vs baseline: 1.1691x; 1.0650x over previous
import functools

import jax
import jax.numpy as jnp
from jax import lax
from jax.experimental import pallas as pl
from jax.experimental.pallas import tpu as pltpu
from jax.experimental.pallas import tpu_sc as plsc

N = 100000
D = 128
ROWS_PER_DEG = 16000
NUM_WORKERS = 32
IW = 128
BS = 4000
_PARAMS = {1: (128, 4), 2: (64, 8), 3: (40, 14), 4: (32, 16),
           5: (24, 22), 6: (16, 32)}
_PARTS = ((1, 2, 3), (4, 5), (6,))


def _reduce(d, R, g, ob):
    def row_body(r, carry):
        base = r * d
        for cc in range(D // 16):
            sl = pl.ds(cc * 16, 16)
            v = g[base, sl]
            for j in range(1, d):
                v = v + g[base + j, sl]
            ob[r, sl] = v
        return carry
    lax.fori_loop(0, R, row_body, 0, unroll=2 if d < 4 else 1)


def _sc_gather_sum(nf, adj_flat, degs):
    mesh = plsc.VectorSubcoreMesh(core_axis_name="c", subcore_axis_name="s")

    degoff, secbase = {}, {}
    ao = so = 0
    for d in degs:
        R, S = _PARAMS[d]
        degoff[d], secbase[d] = ao, so
        ao += ROWS_PER_DEG * d
        so += S * R * d
    idxwords = so

    @functools.partial(
        pl.kernel,
        out_type=jax.ShapeDtypeStruct((len(degs) * ROWS_PER_DEG, D),
                                      jnp.float32),
        mesh=mesh,
        scratch_types=[
            pltpu.VMEM((idxwords,), jnp.int32),
            pltpu.VMEM((IW, D), jnp.float32),
            pltpu.VMEM((IW, D), jnp.float32),
            pltpu.VMEM((IW, D), jnp.float32),
            pltpu.VMEM((IW, D), jnp.float32),
            pltpu.SemaphoreType.DMA,
            pltpu.SemaphoreType.DMA,
            pltpu.SemaphoreType.DMA,
            pltpu.SemaphoreType.DMA,
            pltpu.SemaphoreType.DMA,
        ],
    )
    def k(nf_hbm, adj_hbm, out_hbm, idx_v, g0, g1, o0, o1,
          sem_idx, gs0, gs1, os0, os1):
        cid = lax.axis_index("c")
        sid = lax.axis_index("s")
        wid = sid * 2 + cid
        gbufs, gsems = (g0, g1), (gs0, gs1)
        obufs, osems = (o0, o1), (os0, os1)

        w0s = {}
        slab_cps = []
        for d in degs:
            R, S = _PARAMS[d]
            w0s[d] = jnp.minimum(wid * S * R, ROWS_PER_DEG - S * R)
            n = S * R * d
            slab_cps.append(pltpu.async_copy(
                adj_hbm.at[pl.ds(degoff[d] + w0s[d] * d, n)],
                idx_v.at[pl.ds(secbase[d], n)], sem_idx))
        for cp in slab_cps:
            cp.wait()

        def base_row(d, s):
            R, S = _PARAMS[d]
            return jnp.minimum((wid * S + s) * R, ROWS_PER_DEG - R)

        def idx_slice(d, s):
            R, S = _PARAMS[d]
            off = secbase[d] + (base_row(d, s) - w0s[d]) * d
            return idx_v.at[pl.ds(off, R * d)]

        def store_row0(d, s):
            return degs.index(d) * ROWS_PER_DEG + base_row(d, s)

        for d in degs:
            R, S = _PARAMS[d]
            L = R * d

            if d == 1:
                bufs4 = (g0, g1, o0, o1)
                sems4 = (gs0, gs1, os0, os1)
                cps = [pltpu.async_copy(nf_hbm.at[idx_slice(1, s)],
                                        bufs4[s], sems4[s])
                       for s in range(S)]
                sts = []
                for s in range(S):
                    cps[s].wait()
                    sts.append(pltpu.async_copy(
                        bufs4[s], out_hbm.at[pl.ds(store_row0(1, s), R)],
                        sems4[s]))
                for s in range(S):
                    sts[s].wait()
                continue

            def fire_gather(s, b, d=d, L=L):
                return pltpu.async_copy(
                    nf_hbm.at[idx_slice(d, s)],
                    gbufs[b].at[pl.ds(0, L)], gsems[b])

            def task(s, b, first, last, d=d, R=R, L=L):
                pltpu.make_async_copy(
                    nf_hbm.at[pl.ds(0, L)], gbufs[b].at[pl.ds(0, L)],
                    gsems[b]).wait()
                if not first:
                    pltpu.make_async_copy(
                        obufs[b].at[pl.ds(0, R)],
                        nf_hbm.at[pl.ds(0, R)], osems[b]).wait()
                _reduce(d, R, gbufs[b], obufs[b])
                pltpu.async_copy(
                    obufs[b].at[pl.ds(0, R)],
                    out_hbm.at[pl.ds(store_row0(d, s), R)], osems[b])
                if not last:
                    fire_gather(s + 2, b)

            fire_gather(0, 0)
            fire_gather(1, 1)
            task(0, 0, first=True, last=False)
            task(1, 1, first=True, last=False)

            def outer(o, carry, task=task):
                s0 = 2 * o
                task(s0, 0, first=False, last=False)
                task(s0 + 1, 1, first=False, last=False)
                return carry
            lax.fori_loop(1, S // 2 - 1, outer, 0)

            task(S - 2, 0, first=False, last=True)
            task(S - 1, 1, first=False, last=True)
            for b in range(2):
                pltpu.make_async_copy(
                    obufs[b].at[pl.ds(0, R)],
                    nf_hbm.at[pl.ds(0, R)], osems[b]).wait()

    return k(nf, adj_flat)


def _tc_self(nf, W, b):
    nblocks = N // BS

    def ws_idx(g):
        return (jnp.where(g == 0, 0, 2 * ((g + 3) // 4)), 0, 0)

    def body(x_ref, ws_ref, bs_ref, o_ref):
        o_ref[...] = jnp.dot(
            x_ref[...], ws_ref[0],
            preferred_element_type=jnp.float32) + bs_ref[0, 0]

    br = b.reshape(b.shape[0], 1, D)
    return pl.pallas_call(
        body,
        grid=(nblocks,),
        in_specs=[
            pl.BlockSpec((BS, D), lambda g: (g, 0)),
            pl.BlockSpec((1, D, D), ws_idx),
            pl.BlockSpec((1, 1, D), ws_idx),
        ],
        out_specs=pl.BlockSpec((BS, D), lambda g: (g, 0)),
        out_shape=jax.ShapeDtypeStruct((N, D), jnp.float32),
    )(nf, W, br)


def _tc_add_neigh(prev, nsum, W, b, blk0):
    nblocks = nsum.shape[0] // BS

    def wn_idx(g):
        return (2 * ((g + blk0 + 3) // 4) - 1, 0, 0)

    def body(prev_ref, ns_ref, wn_ref, bn_ref, o_ref):
        o_ref[...] = prev_ref[...] + jnp.dot(
            ns_ref[...], wn_ref[0],
            preferred_element_type=jnp.float32) + bn_ref[0, 0]

    br = b.reshape(b.shape[0], 1, D)
    return pl.pallas_call(
        body,
        grid=(nblocks,),
        in_specs=[
            pl.BlockSpec((BS, D), lambda g: (g + blk0, 0)),
            pl.BlockSpec((BS, D), lambda g: (g, 0)),
            pl.BlockSpec((1, D, D), wn_idx),
            pl.BlockSpec((1, 1, D), wn_idx),
        ],
        out_specs=pl.BlockSpec((BS, D), lambda g: (g + blk0, 0)),
        out_shape=jax.ShapeDtypeStruct((N, D), jnp.float32),
        input_output_aliases={0: 0},
    )(prev, nsum, W, br)


def kernel(node_features, deg_slice, deg_adj_1, deg_adj_2, deg_adj_3,
           deg_adj_4, deg_adj_5, deg_adj_6, W, b):
    adjs = {1: deg_adj_1, 2: deg_adj_2, 3: deg_adj_3,
            4: deg_adj_4, 5: deg_adj_5, 6: deg_adj_6}

    def flat(degs):
        parts = [adjs[d] if adjs[d].dtype == jnp.int32
                 else adjs[d].astype(jnp.int32) for d in degs]
        return jnp.concatenate([p.reshape(-1) for p in parts])

    out_self = _tc_self(node_features, W, b)
    nsums = [_sc_gather_sum(node_features, flat(p), p) for p in _PARTS]
    out = out_self
    for part, nsum in reversed(list(zip(_PARTS, nsums))):
        out = _tc_add_neigh(out, nsum, W, b, blk0=1 + 4 * (part[0] - 1))
    return out
